# baseline (speedup 1.0000x reference)
import functools

import jax
import jax.numpy as jnp
from jax import lax
from jax.experimental import pallas as pl
from jax.experimental.pallas import tpu as pltpu
from jax.experimental.pallas import tpu_sc as plsc

N, E, D, R, NB = 10000, 320000, 128, 8, 8

NC, NS = 2, 16
NW = NC * NS
EW = E // NW
SUB = 100
ROWS_W = EW // SUB
GR = 10
NGR = ROWS_W // GR
NBUF = 3
RPT = 624
TAIL = N - NS * RPT
ZR = 16
BN = 2000


def _sc_edge_body(table, gidx, dstr, out, gidx_v, dst_v, rows_v, hacc,
                  sem_i, sem_z, sem_g0, sem_g1, sem_g2, sem_g3, sem_s0,
                  sem_s1, sem_s2, sem_s3):
    c = lax.axis_index("c")
    s = lax.axis_index("s")
    wid = s * NC + c

    icps = [pltpu.async_copy(gidx.at[wid, 0], gidx_v.at[0], sem_i),
            pltpu.async_copy(dstr.at[wid, 0], dst_v.at[0], sem_i)]

    zvec = jnp.zeros((16,), jnp.float32)

    def zbuf(i, carry):
        for j in range(D // 16):
            rows_v[NBUF - 1, i, pl.ds(j * 16, 16)] = zvec
        return carry

    lax.fori_loop(0, ZR, zbuf, 0)
    zero_v = rows_v.at[NBUF - 1, pl.ds(0, ZR)]
    zcps = [pltpu.async_copy(zero_v, hacc.at[pl.ds(s * RPT + k * ZR, ZR)],
                             sem_z) for k in range(RPT // ZR)]

    @pl.when(s == NS - 1)
    def _():
        pltpu.async_copy(zero_v, hacc.at[pl.ds(NS * RPT, TAIL)], sem_z).wait()

    for cp in icps:
        cp.wait()
    icps = None

    semg = (sem_g0, sem_g1, sem_g2, sem_g3)
    sems = (sem_s0, sem_s1, sem_s2, sem_s3)

    def fire_g(j):
        g, r = divmod(j, GR)
        return pltpu.async_copy(table.at[gidx_v.at[g & 1, r]],
                                rows_v.at[j % NBUF], semg[j % NBUF])

    def fire_s(j):
        g, r = divmod(j, GR)
        return pltpu.async_copy(rows_v.at[j % NBUF],
                                hacc.at[dst_v.at[g & 1, r]],
                                sems[j % NBUF], add=True)

    gcps = {}
    scps = {}
    for j in range(ROWS_W):
        g, r = divmod(j, GR)
        if j == 2:
            for cp in zcps:
                cp.wait()
            plsc.subcore_barrier()
        if r == 0 and g > 0:
            for cp in icps:
                cp.wait()
        if j >= NBUF:
            scps[j - NBUF].wait()
        gcps[j] = fire_g(j)
        if j >= 2:
            gcps[j - 2].wait()
            scps[j - 2] = fire_s(j - 2)
        if r == NBUF - 1 and g + 1 < NGR:
            q = (g + 1) & 1
            icps = [pltpu.async_copy(gidx.at[wid, g + 1], gidx_v.at[q],
                                     sem_i),
                    pltpu.async_copy(dstr.at[wid, g + 1], dst_v.at[q],
                                     sem_i)]
    for j in (ROWS_W - 2, ROWS_W - 1):
        gcps[j].wait()
        scps[j] = fire_s(j)
    for j in range(ROWS_W - NBUF, ROWS_W):
        scps[j].wait()
    plsc.subcore_barrier()

    pltpu.sync_copy(hacc.at[pl.ds(s * RPT, RPT)],
                    out.at[pl.ds(c * N + s * RPT, RPT)])

    @pl.when(s == NS - 1)
    def _():
        pltpu.sync_copy(hacc.at[pl.ds(NS * RPT, TAIL)],
                        out.at[pl.ds(c * N + NS * RPT, TAIL)])


_SC_EDGE_CACHE = []


def _sc_edge(table, gidx, dstr):
    if not _SC_EDGE_CACHE:
        _SC_EDGE_CACHE.append(functools.partial(
            pl.kernel,
            mesh=plsc.VectorSubcoreMesh(
                core_axis_name="c", subcore_axis_name="s", num_cores=NC),
            out_type=jax.ShapeDtypeStruct((NC * N, D), jnp.float32),
            scratch_types=[
                pltpu.VMEM((2, GR, SUB), jnp.int32),
                pltpu.VMEM((2, GR, SUB), jnp.int32),
                pltpu.VMEM((NBUF, SUB, D), jnp.float32),
                pltpu.VMEM_SHARED((N, D), jnp.float32),
            ] + [pltpu.SemaphoreType.DMA] * 10,
        )(_sc_edge_body))
    return _SC_EDGE_CACHE[0](table, gidx, dstr)


def _wr(bases_ref, coeff_ref, r):
    w = coeff_ref[r, 0] * bases_ref[0]
    for b in range(1, NB):
        w = w + coeff_ref[r, b] * bases_ref[b]
    return w


def _tc_first_body(x_ref, bases_ref, coeff_ref, out_ref):
    r = pl.program_id(1)
    a = x_ref[...]
    w = _wr(bases_ref, coeff_ref, r)
    out_ref[...] = jnp.dot(a.astype(jnp.bfloat16), w.astype(jnp.bfloat16),
                           preferred_element_type=jnp.float32)


def _tc_mid_body(p0_ref, p1_ref, bias_ref, bases_ref, coeff_ref, out_ref):
    r = pl.program_id(1)
    a = jnp.maximum(p0_ref[...] + p1_ref[...] + bias_ref[...], 0.0)
    w = _wr(bases_ref, coeff_ref, r)
    out_ref[...] = jnp.dot(a.astype(jnp.bfloat16), w.astype(jnp.bfloat16),
                           preferred_element_type=jnp.float32)


_ACT_SPECS = [
    pl.BlockSpec((NB, D, D), lambda i, r: (0, 0, 0)),
    pl.BlockSpec((R, NB), lambda i, r: (0, 0)),
]
_OUT_SPEC = pl.BlockSpec((BN, D), lambda i, r: (r * (N // BN) + i, 0))


def _tc_first(x, bases, coeff):
    return pl.pallas_call(
        _tc_first_body,
        grid=(N // BN, R),
        in_specs=[pl.BlockSpec((BN, D), lambda i, r: (i, 0))] + _ACT_SPECS,
        out_specs=_OUT_SPEC,
        out_shape=jax.ShapeDtypeStruct((R * N, D), jnp.float32),
    )(x, bases, coeff)


def _tc_mid(parts, bias, bases, coeff):
    return pl.pallas_call(
        _tc_mid_body,
        grid=(N // BN, R),
        in_specs=[
            pl.BlockSpec((BN, D), lambda i, r: (i, 0)),
            pl.BlockSpec((BN, D), lambda i, r: (N // BN + i, 0)),
            pl.BlockSpec((1, D), lambda i, r: (0, 0)),
        ] + _ACT_SPECS,
        out_specs=_OUT_SPEC,
        out_shape=jax.ShapeDtypeStruct((R * N, D), jnp.float32),
    )(parts, parts, bias, bases, coeff)


def _pool_body(p0_ref, p1_ref, bias_ref, gw_ref, gb_ref, out_ref,
               m_ref, den_ref, num_ref):
    j = pl.program_id(0)
    nb = pl.num_programs(0)
    h = p0_ref[...] + p1_ref[...] + bias_ref[...]
    g = jnp.sum(h * gw_ref[...], axis=1, keepdims=True) + gb_ref[...]
    bm = jnp.max(g).reshape(1, 1)

    @pl.when(j == 0)
    def _():
        e = jnp.exp(g - bm)
        m_ref[...] = bm
        num_ref[...] = jnp.sum(h * e, axis=0, keepdims=True)
        den_ref[...] = jnp.sum(e).reshape(1, 1)

    @pl.when(j > 0)
    def _():
        m_new = jnp.maximum(m_ref[...], bm)
        alpha = jnp.exp(m_ref[...] - m_new)
        e = jnp.exp(g - m_new)
        num_ref[...] = num_ref[...] * alpha + jnp.sum(h * e, axis=0,
                                                      keepdims=True)
        den_ref[...] = den_ref[...] * alpha + jnp.sum(e).reshape(1, 1)
        m_ref[...] = m_new

    @pl.when(j == nb - 1)
    def _():
        out_ref[...] = num_ref[...] / den_ref[...]


def _pool(parts, bias, gw_row, gb):
    return pl.pallas_call(
        _pool_body,
        grid=(N // BN,),
        in_specs=[
            pl.BlockSpec((BN, D), lambda j: (j, 0)),
            pl.BlockSpec((BN, D), lambda j: (N // BN + j, 0)),
            pl.BlockSpec((1, D), lambda j: (0, 0)),
            pl.BlockSpec((1, D), lambda j: (0, 0)),
            pl.BlockSpec((1, 1), lambda j: (0, 0)),
        ],
        out_specs=pl.BlockSpec((1, D), lambda j: (0, 0)),
        out_shape=jax.ShapeDtypeStruct((1, D), jnp.float32),
        scratch_shapes=[
            pltpu.VMEM((1, 1), jnp.float32),
            pltpu.VMEM((1, 1), jnp.float32),
            pltpu.VMEM((1, D), jnp.float32),
        ],
    )(parts, parts, bias, gw_row, gb)


def kernel(x, edge_index, etype, bases0, coeff0, bias0, bases1, coeff1, bias1,
           bases2, coeff2, bias2, gate_w, gate_b):
    src = edge_index[0]
    dst = edge_index[1]
    gidx = (etype * N + src).reshape(NW, NGR, GR, SUB)
    dstr = dst.reshape(NW, NGR, GR, SUB)
    b0 = bias0.reshape(1, D)
    b1 = bias1.reshape(1, D)
    b2 = bias2.reshape(1, D)

    t0 = _tc_first(x, bases0, coeff0)
    parts = _sc_edge(t0, gidx, dstr)
    t1 = _tc_mid(parts, b0, bases1, coeff1)
    parts = _sc_edge(t1, gidx, dstr)
    t2 = _tc_mid(parts, b1, bases2, coeff2)
    parts = _sc_edge(t2, gidx, dstr)
    return _pool(parts, b2, gate_w.reshape(1, D), gate_b.reshape(1, 1))


# speedup vs baseline: 1.0160x; 1.0160x over previous
import functools

import jax
import jax.numpy as jnp
from jax import lax
from jax.experimental import pallas as pl
from jax.experimental.pallas import tpu as pltpu
from jax.experimental.pallas import tpu_sc as plsc

N, E, D, R, NB = 10000, 320000, 128, 8, 8

NC, NS = 2, 16
NW = NC * NS
EW = E // NW
SUB = 100
ROWS_W = EW // SUB
GR = 10
NGR = ROWS_W // GR
NBUF = 3
RPT = 624
TAIL = N - NS * RPT
ZR = 16
BN = 2000


def _sc_edge_body(table, gidx, dstr, out, gidx_v, dst_v, rows_v, hacc,
                  sem_i, sem_z, sem_g0, sem_g1, sem_g2, sem_g3, sem_s0,
                  sem_s1, sem_s2, sem_s3):
    c = lax.axis_index("c")
    s = lax.axis_index("s")
    wid = s * NC + c

    icps = [pltpu.async_copy(gidx.at[wid, 0], gidx_v.at[0], sem_i),
            pltpu.async_copy(dstr.at[wid, 0], dst_v.at[0], sem_i)]

    zvec = jnp.zeros((16,), jnp.float32)

    def zbuf(i, carry):
        for j in range(D // 16):
            rows_v[NBUF - 1, i, pl.ds(j * 16, 16)] = zvec
        return carry

    lax.fori_loop(0, ZR, zbuf, 0)
    zero_v = rows_v.at[NBUF - 1, pl.ds(0, ZR)]
    zcps = [pltpu.async_copy(zero_v, hacc.at[pl.ds(s * RPT + k * ZR, ZR)],
                             sem_z) for k in range(RPT // ZR)]

    @pl.when(s == NS - 1)
    def _():
        pltpu.async_copy(zero_v, hacc.at[pl.ds(NS * RPT, TAIL)], sem_z).wait()

    for cp in icps:
        cp.wait()
    icps = None

    semg = (sem_g0, sem_g1, sem_g2, sem_g3)
    sems = (sem_s0, sem_s1, sem_s2, sem_s3)

    def fire_g(j):
        g, r = divmod(j, GR)
        return pltpu.async_copy(table.at[gidx_v.at[g & 1, r]],
                                rows_v.at[j % NBUF], semg[j % NBUF])

    def fire_s(j):
        g, r = divmod(j, GR)
        return pltpu.async_copy(rows_v.at[j % NBUF],
                                hacc.at[dst_v.at[g & 1, r]],
                                sems[j % NBUF], add=True)

    gcps = {}
    scps = {}
    for j in range(ROWS_W):
        g, r = divmod(j, GR)
        if j == 2:
            for cp in zcps:
                cp.wait()
            plsc.subcore_barrier()
        if r == 0 and g > 0:
            for cp in icps:
                cp.wait()
        if j >= NBUF:
            scps[j - NBUF].wait()
        gcps[j] = fire_g(j)
        if j >= 2:
            gcps[j - 2].wait()
            scps[j - 2] = fire_s(j - 2)
        if r == NBUF - 1 and g + 1 < NGR:
            q = (g + 1) & 1
            icps = [pltpu.async_copy(gidx.at[wid, g + 1], gidx_v.at[q],
                                     sem_i),
                    pltpu.async_copy(dstr.at[wid, g + 1], dst_v.at[q],
                                     sem_i)]
    for j in (ROWS_W - 2, ROWS_W - 1):
        gcps[j].wait()
        scps[j] = fire_s(j)
    for j in range(ROWS_W - NBUF, ROWS_W):
        scps[j].wait()
    plsc.subcore_barrier()

    pltpu.sync_copy(hacc.at[pl.ds(s * RPT, RPT)],
                    out.at[pl.ds(c * N + s * RPT, RPT)])

    @pl.when(s == NS - 1)
    def _():
        pltpu.sync_copy(hacc.at[pl.ds(NS * RPT, TAIL)],
                        out.at[pl.ds(c * N + NS * RPT, TAIL)])


_SC_EDGE_CACHE = []


def _sc_edge(table, gidx, dstr):
    if not _SC_EDGE_CACHE:
        _SC_EDGE_CACHE.append(functools.partial(
            pl.kernel,
            mesh=plsc.VectorSubcoreMesh(
                core_axis_name="c", subcore_axis_name="s", num_cores=NC),
            out_type=jax.ShapeDtypeStruct((NC * N, D), jnp.float32),
            scratch_types=[
                pltpu.VMEM((2, GR, SUB), jnp.int32),
                pltpu.VMEM((2, GR, SUB), jnp.int32),
                pltpu.VMEM((NBUF, SUB, D), jnp.float32),
                pltpu.VMEM_SHARED((N, D), jnp.float32),
            ] + [pltpu.SemaphoreType.DMA] * 10,
        )(_sc_edge_body))
    return _SC_EDGE_CACHE[0](table, gidx, dstr)


def _wr(bases_ref, coeff_ref, r):
    w = coeff_ref[r, 0] * bases_ref[0]
    for b in range(1, NB):
        w = w + coeff_ref[r, b] * bases_ref[b]
    return w


def _tc_first_body(x_ref, bases_ref, coeff_ref, out_ref):
    r = pl.program_id(1)
    a = x_ref[...]
    w = _wr(bases_ref, coeff_ref, r)
    out_ref[...] = jnp.dot(a, w, preferred_element_type=jnp.float32)


def _tc_mid_body(p0_ref, p1_ref, bias_ref, bases_ref, coeff_ref, out_ref):
    r = pl.program_id(1)
    a = jnp.maximum(p0_ref[...] + p1_ref[...] + bias_ref[...], 0.0)
    w = _wr(bases_ref, coeff_ref, r)
    out_ref[...] = jnp.dot(a, w, preferred_element_type=jnp.float32)


_ACT_SPECS = [
    pl.BlockSpec((NB, D, D), lambda i, r: (0, 0, 0)),
    pl.BlockSpec((R, NB), lambda i, r: (0, 0)),
]
_OUT_SPEC = pl.BlockSpec((BN, D), lambda i, r: (r * (N // BN) + i, 0))


def _tc_first(x, bases, coeff):
    return pl.pallas_call(
        _tc_first_body,
        grid=(N // BN, R),
        in_specs=[pl.BlockSpec((BN, D), lambda i, r: (i, 0))] + _ACT_SPECS,
        out_specs=_OUT_SPEC,
        out_shape=jax.ShapeDtypeStruct((R * N, D), jnp.float32),
    )(x, bases, coeff)


def _tc_mid(parts, bias, bases, coeff):
    return pl.pallas_call(
        _tc_mid_body,
        grid=(N // BN, R),
        in_specs=[
            pl.BlockSpec((BN, D), lambda i, r: (i, 0)),
            pl.BlockSpec((BN, D), lambda i, r: (N // BN + i, 0)),
            pl.BlockSpec((1, D), lambda i, r: (0, 0)),
        ] + _ACT_SPECS,
        out_specs=_OUT_SPEC,
        out_shape=jax.ShapeDtypeStruct((R * N, D), jnp.float32),
    )(parts, parts, bias, bases, coeff)


def _pool_body(p0_ref, p1_ref, bias_ref, gw_ref, gb_ref, out_ref,
               m_ref, den_ref, num_ref):
    j = pl.program_id(0)
    nb = pl.num_programs(0)
    h = p0_ref[...] + p1_ref[...] + bias_ref[...]
    g = jnp.sum(h * gw_ref[...], axis=1, keepdims=True) + gb_ref[...]
    bm = jnp.max(g).reshape(1, 1)

    @pl.when(j == 0)
    def _():
        e = jnp.exp(g - bm)
        m_ref[...] = bm
        num_ref[...] = jnp.sum(h * e, axis=0, keepdims=True)
        den_ref[...] = jnp.sum(e).reshape(1, 1)

    @pl.when(j > 0)
    def _():
        m_new = jnp.maximum(m_ref[...], bm)
        alpha = jnp.exp(m_ref[...] - m_new)
        e = jnp.exp(g - m_new)
        num_ref[...] = num_ref[...] * alpha + jnp.sum(h * e, axis=0,
                                                      keepdims=True)
        den_ref[...] = den_ref[...] * alpha + jnp.sum(e).reshape(1, 1)
        m_ref[...] = m_new

    @pl.when(j == nb - 1)
    def _():
        out_ref[...] = num_ref[...] / den_ref[...]


def _pool(parts, bias, gw_row, gb):
    return pl.pallas_call(
        _pool_body,
        grid=(N // BN,),
        in_specs=[
            pl.BlockSpec((BN, D), lambda j: (j, 0)),
            pl.BlockSpec((BN, D), lambda j: (N // BN + j, 0)),
            pl.BlockSpec((1, D), lambda j: (0, 0)),
            pl.BlockSpec((1, D), lambda j: (0, 0)),
            pl.BlockSpec((1, 1), lambda j: (0, 0)),
        ],
        out_specs=pl.BlockSpec((1, D), lambda j: (0, 0)),
        out_shape=jax.ShapeDtypeStruct((1, D), jnp.float32),
        scratch_shapes=[
            pltpu.VMEM((1, 1), jnp.float32),
            pltpu.VMEM((1, 1), jnp.float32),
            pltpu.VMEM((1, D), jnp.float32),
        ],
    )(parts, parts, bias, gw_row, gb)


def kernel(x, edge_index, etype, bases0, coeff0, bias0, bases1, coeff1, bias1,
           bases2, coeff2, bias2, gate_w, gate_b):
    src = edge_index[0]
    dst = edge_index[1]
    gidx = (etype * N + src).reshape(NW, NGR, GR, SUB)
    dstr = dst.reshape(NW, NGR, GR, SUB)
    b0 = bias0.reshape(1, D)
    b1 = bias1.reshape(1, D)
    b2 = bias2.reshape(1, D)

    t0 = _tc_first(x, bases0, coeff0)
    parts = _sc_edge(t0, gidx, dstr)
    t1 = _tc_mid(parts, b0, bases1, coeff1)
    parts = _sc_edge(t1, gidx, dstr)
    t2 = _tc_mid(parts, b1, bases2, coeff2)
    parts = _sc_edge(t2, gidx, dstr)
    return _pool(parts, b2, gate_w.reshape(1, D), gate_b.reshape(1, 1))
